# baseline (speedup 1.0000x reference)
import functools

import jax
import jax.numpy as jnp
from jax import lax
from jax.experimental import pallas as pl
from jax.experimental.pallas import tpu as pltpu
from jax.experimental.pallas import tpu_sc as plsc

DIM = 64
BATCH = 16384
NC = 2
NS = 16
L = 16
NW = NC * NS
BPW = BATCH // NW
CHUNK = 128
NCHUNK = BPW // CHUNK
NBUF = 3
WIDE = 2 * DIM

_mesh = plsc.VectorSubcoreMesh(core_axis_name="c", subcore_axis_name="s")

_scratch = [
    pltpu.VMEM((BPW,), jnp.int32),
    pltpu.VMEM((BPW,), jnp.int32),
    pltpu.VMEM((BPW,), jnp.int32),
    pltpu.VMEM((BPW,), jnp.int32),
    pltpu.VMEM((BPW,), jnp.int32),
    pltpu.VMEM((BPW,), jnp.float32),
    pltpu.VMEM((BPW,), jnp.float32),
    pltpu.VMEM((BPW,), jnp.float32),
]
_scratch += [pltpu.VMEM((CHUNK, WIDE), jnp.float32) for _ in range(2 * NBUF)]
_scratch += [pltpu.VMEM((CHUNK * L,), jnp.float32) for _ in range(4)]
_scratch += [pltpu.SemaphoreType.DMA for _ in range(NBUF + 1)]


@functools.partial(
    pl.kernel,
    out_type=jax.ShapeDtypeStruct((BATCH,), jnp.float32),
    mesh=_mesh,
    compiler_params=pltpu.CompilerParams(
        needs_layout_passes=False, use_tc_tiling_on_sc=True),
    scratch_types=_scratch,
)
def _glove_sc(i_idx, j_idx, wi2, wj2, bi_flat, bj_flat, out_hbm,
              idx_i, idx_j, ihalf, jhalf, sel, bias_i, bias_j, out_v,
              ri0, ri1, ri2, rj0, rj1, rj2, p00, p01, p10, p11,
              sem0, sem1, sem2, sem_b):
    ri = (ri0, ri1, ri2)
    rj = (rj0, rj1, rj2)
    sems = (sem0, sem1, sem2)
    wid = lax.axis_index("s") * NC + lax.axis_index("c")
    base = wid * BPW

    pltpu.sync_copy(i_idx.at[pl.ds(base, BPW)], idx_i)
    pltpu.sync_copy(j_idx.at[pl.ds(base, BPW)], idx_j)

    def split_body(t, carry):
        o = pl.ds(pl.multiple_of(t * L, L), L)
        ii = idx_i[o]
        jj = idx_j[o]
        ihalf[o] = lax.shift_right_logical(ii, 1)
        jhalf[o] = lax.shift_right_logical(jj, 1)
        sel[o] = (ii & 1) * 2 + (jj & 1)
        return carry

    lax.fori_loop(0, BPW // L, split_body, 0)

    bias_handles = []
    row_handles = {}

    def fire(c):
        b = c % NBUF
        rows = pl.ds(c * CHUNK, CHUNK)
        row_handles[c] = (
            pltpu.async_copy(wi2.at[ihalf.at[rows]], ri[b], sems[b]),
            pltpu.async_copy(wj2.at[jhalf.at[rows]], rj[b], sems[b]),
        )

    for c in range(NCHUNK):
        rows = pl.ds(c * CHUNK, CHUNK)
        bias_handles.append(
            pltpu.async_copy(bi_flat.at[idx_i.at[rows]], bias_i.at[rows],
                             sem_b))
        bias_handles.append(
            pltpu.async_copy(bj_flat.at[idx_j.at[rows]], bias_j.at[rows],
                             sem_b))
    fire(0)
    fire(1)
    for h in bias_handles:
        h.wait()

    iota = lax.iota(jnp.int32, L)

    for c in range(NCHUNK):
        b = c % NBUF
        for h in row_handles.pop(c):
            h.wait()
        if c + 2 < NCHUNK:
            fire(c + 2)

        a_ref = ri[b]
        b_ref = rj[b]

        def body1(t, carry):
            av = [a_ref[t, pl.ds(k * L, L)] for k in range(WIDE // L)]
            bv = [b_ref[t, pl.ds(k * L, L)] for k in range(WIDE // L)]
            nh = DIM // L
            m00 = av[0] * bv[0]
            m01 = av[0] * bv[nh]
            m10 = av[nh] * bv[0]
            m11 = av[nh] * bv[nh]
            for k in range(1, nh):
                m00 += av[k] * bv[k]
                m01 += av[k] * bv[nh + k]
                m10 += av[nh + k] * bv[k]
                m11 += av[nh + k] * bv[nh + k]
            o = pl.ds(pl.multiple_of(t * L, L), L)
            p00[o] = m00
            p01[o] = m01
            p10[o] = m10
            p11[o] = m11
            return carry

        lax.fori_loop(0, CHUNK, body1, 0)

        def body2(g, carry):
            flat = g * (L * L) + iota * L
            a00 = plsc.load_gather(p00, [flat])
            a01 = plsc.load_gather(p01, [flat])
            a10 = plsc.load_gather(p10, [flat])
            a11 = plsc.load_gather(p11, [flat])
            for k in range(1, L):
                a00 += plsc.load_gather(p00, [flat + k])
                a01 += plsc.load_gather(p01, [flat + k])
                a10 += plsc.load_gather(p10, [flat + k])
                a11 += plsc.load_gather(p11, [flat + k])
            o = pl.ds(pl.multiple_of(c * CHUNK + g * L, L), L)
            sv = sel[o]
            res = jnp.where(sv == 0, a00,
                            jnp.where(sv == 1, a01,
                                      jnp.where(sv == 2, a10, a11)))
            out_v[o] = res + bias_i[o] + bias_j[o]
            return carry

        lax.fori_loop(0, CHUNK // L, body2, 0)

    pltpu.sync_copy(out_v, out_hbm.at[pl.ds(base, BPW)])


def kernel(i_idx, j_idx, wi, wj, bi, bj):
    wi2 = wi.reshape(wi.shape[0] // 2, WIDE)
    wj2 = wj.reshape(wj.shape[0] // 2, WIDE)
    return _glove_sc(i_idx.astype(jnp.int32), j_idx.astype(jnp.int32),
                     wi2, wj2, bi.reshape(-1), bj.reshape(-1))


# speedup vs baseline: 1.0512x; 1.0512x over previous
import functools

import jax
import jax.numpy as jnp
from jax import lax
from jax.experimental import pallas as pl
from jax.experimental.pallas import tpu as pltpu
from jax.experimental.pallas import tpu_sc as plsc

DIM = 64
BATCH = 16384
NC = 2
NS = 16
L = 16
NW = NC * NS
BPW = BATCH // NW

_mesh = plsc.VectorSubcoreMesh(core_axis_name="c", subcore_axis_name="s")


@functools.partial(
    pl.kernel,
    out_type=jax.ShapeDtypeStruct((BATCH,), jnp.float32),
    mesh=_mesh,
    compiler_params=pltpu.CompilerParams(
        needs_layout_passes=False, use_tc_tiling_on_sc=False),
    scratch_types=[
        pltpu.VMEM((BPW,), jnp.int32),
        pltpu.VMEM((BPW,), jnp.int32),
        pltpu.VMEM((BPW, DIM), jnp.float32),
        pltpu.VMEM((BPW, DIM), jnp.float32),
        pltpu.VMEM((BPW,), jnp.float32),
        pltpu.VMEM((BPW,), jnp.float32),
        pltpu.VMEM((BPW * L,), jnp.float32),
        pltpu.VMEM((BPW,), jnp.float32),
        pltpu.SemaphoreType.DMA,
    ],
)
def _glove_sc(i_idx, j_idx, wi, wj, bi_flat, bj_flat, out_hbm,
              idx_i, idx_j, rows_i, rows_j, bias_i, bias_j, partial,
              out_v, sem):
    wid = lax.axis_index("s") * NC + lax.axis_index("c")
    base = wid * BPW

    pltpu.sync_copy(i_idx.at[pl.ds(base, BPW)], idx_i)
    pltpu.sync_copy(j_idx.at[pl.ds(base, BPW)], idx_j)

    handles = [
        pltpu.async_copy(wi.at[idx_i], rows_i, sem),
        pltpu.async_copy(wj.at[idx_j], rows_j, sem),
        pltpu.async_copy(bi_flat.at[idx_i], bias_i, sem),
        pltpu.async_copy(bj_flat.at[idx_j], bias_j, sem),
    ]
    for h in handles:
        h.wait()

    def body1(t, carry):
        acc = rows_i[t, pl.ds(0, L)] * rows_j[t, pl.ds(0, L)]
        for k in range(1, DIM // L):
            acc += rows_i[t, pl.ds(k * L, L)] * rows_j[t, pl.ds(k * L, L)]
        partial[pl.ds(pl.multiple_of(t * L, L), L)] = acc
        return carry

    lax.fori_loop(0, BPW, body1, 0)

    iota = lax.iota(jnp.int32, L)

    def body2(g, carry):
        flat = g * (L * L) + iota * L
        acc = plsc.load_gather(partial, [flat])
        for k in range(1, L):
            acc += plsc.load_gather(partial, [flat + k])
        o = pl.ds(pl.multiple_of(g * L, L), L)
        out_v[o] = acc + bias_i[o] + bias_j[o]
        return carry

    lax.fori_loop(0, BPW // L, body2, 0)

    pltpu.sync_copy(out_v, out_hbm.at[pl.ds(base, BPW)])


def kernel(i_idx, j_idx, wi, wj, bi, bj):
    return _glove_sc(i_idx.astype(jnp.int32), j_idx.astype(jnp.int32),
                     wi, wj, bi.reshape(-1), bj.reshape(-1))
